# baseline (speedup 1.0000x reference)
import functools

import jax
import jax.numpy as jnp
from jax import lax
from jax.experimental import pallas as pl
from jax.experimental.pallas import tpu as pltpu
from jax.experimental.pallas import tpu_sc as plsc

NUM_CORES = 2
NUM_SUBCORES = 16
NUM_WORKERS = NUM_CORES * NUM_SUBCORES

B = 4096 * 200
D = 64

STREAM = 128
G = 8
GROUP = G * STREAM

B_PER_W = B // NUM_WORKERS
N_GROUPS = B_PER_W // GROUP
ROWS_PER_W = B_PER_W // STREAM

_mesh = plsc.VectorSubcoreMesh(core_axis_name="c", subcore_axis_name="s")


@functools.partial(
    pl.kernel,
    mesh=_mesh,
    out_type=jax.ShapeDtypeStruct((B, D), jnp.float32),
    scratch_types=[
        pltpu.VMEM((G, STREAM), jnp.int32),
        pltpu.VMEM((GROUP, D), jnp.float32),
        pltpu.SemaphoreType.DMA,
        pltpu.SemaphoreType.DMA,
    ],
    compiler_params=pltpu.CompilerParams(use_tc_tiling_on_sc=False),
)
def _emb_lookup(idx_hbm, table_hbm, out_hbm, idx_v, rows_v, idx_sem, gat_sem):
    wid = lax.axis_index("s") * NUM_CORES + lax.axis_index("c")
    row_base = wid * ROWS_PER_W
    out_base = wid * B_PER_W

    def body(g, _):
        pltpu.async_copy(
            idx_hbm.at[pl.ds(row_base + g * G, G)],
            idx_v.at[...],
            idx_sem,
        ).wait()
        handles = []
        for j in range(G):
            handles.append(
                pltpu.async_copy(
                    table_hbm.at[idx_v.at[j]],
                    rows_v.at[pl.ds(j * STREAM, STREAM)],
                    gat_sem,
                )
            )
        for h in handles:
            h.wait()
        pltpu.sync_copy(rows_v, out_hbm.at[pl.ds(out_base + g * GROUP, GROUP)])
        return ()

    lax.fori_loop(0, N_GROUPS, body, (), unroll=False)


def kernel(sentences_indices, table):
    idx2d = sentences_indices.reshape(B // STREAM, STREAM).astype(jnp.int32)
    out = _emb_lookup(idx2d, table)
    return out.reshape(sentences_indices.shape + (D,))


# speedup vs baseline: 1.0126x; 1.0126x over previous
import functools

import jax
import jax.numpy as jnp
from jax import lax
from jax.experimental import pallas as pl
from jax.experimental.pallas import tpu as pltpu
from jax.experimental.pallas import tpu_sc as plsc

NUM_CORES = 2
NUM_SUBCORES = 16
NUM_WORKERS = NUM_CORES * NUM_SUBCORES

B = 4096 * 200
D = 64

STREAM = 128
G = 5
GROUP = G * STREAM

NBUF = 2

B_PER_W = B // NUM_WORKERS
N_GROUPS = B_PER_W // GROUP
ROWS_PER_W = B_PER_W // STREAM

_mesh = plsc.VectorSubcoreMesh(core_axis_name="c", subcore_axis_name="s")


@functools.partial(
    pl.kernel,
    mesh=_mesh,
    out_type=jax.ShapeDtypeStruct((B, D), jnp.float32),
    scratch_types=[
        pltpu.VMEM((ROWS_PER_W, STREAM), jnp.int32),
        pltpu.VMEM((GROUP, D), jnp.float32),
        pltpu.VMEM((GROUP, D), jnp.float32),
        pltpu.SemaphoreType.DMA,
        pltpu.SemaphoreType.DMA,
        pltpu.SemaphoreType.DMA,
        pltpu.SemaphoreType.DMA,
    ],
    compiler_params=pltpu.CompilerParams(use_tc_tiling_on_sc=False),
)
def _emb_lookup(idx_hbm, table_hbm, out_hbm, idx_v, rows0, rows1,
                gat_sem0, gat_sem1, wb_sem0, wb_sem1):
    rows = (rows0, rows1)
    gat_sem = (gat_sem0, gat_sem1)
    wb_sem = (wb_sem0, wb_sem1)

    wid = lax.axis_index("s") * NUM_CORES + lax.axis_index("c")
    out_base = wid * B_PER_W

    pltpu.async_copy(
        idx_hbm.at[pl.ds(wid * ROWS_PER_W, ROWS_PER_W)], idx_v, gat_sem0
    ).wait()

    def fire(g, b):
        for j in range(G):
            pltpu.async_copy(
                table_hbm.at[idx_v.at[g * G + j]],
                rows[b].at[pl.ds(j * STREAM, STREAM)],
                gat_sem[b],
            )

    def drain_gathers(g, b):
        for j in range(G):
            pltpu.make_async_copy(
                table_hbm.at[idx_v.at[g * G + j]],
                rows[b].at[pl.ds(j * STREAM, STREAM)],
                gat_sem[b],
            ).wait()

    for b in range(NBUF):
        fire(b, b)

    def body(k, _):
        for b in range(NBUF):
            g_done = k * NBUF + b
            drain_gathers(g_done, b)
            h = pltpu.async_copy(
                rows[b], out_hbm.at[pl.ds(out_base + g_done * GROUP, GROUP)],
                wb_sem[b],
            )
            h.wait()
            fire(g_done + NBUF, b)
        return ()

    lax.fori_loop(0, N_GROUPS // NBUF - 1, body, (), unroll=False)

    for b in range(NBUF):
        g_done = N_GROUPS - NBUF + b
        drain_gathers(g_done, b)
        pltpu.async_copy(
            rows[b], out_hbm.at[pl.ds(out_base + g_done * GROUP, GROUP)],
            wb_sem[b],
        ).wait()


def kernel(sentences_indices, table):
    idx2d = sentences_indices.reshape(B // STREAM, STREAM).astype(jnp.int32)
    out = _emb_lookup(idx2d, table)
    return out.reshape(sentences_indices.shape + (D,))


# speedup vs baseline: 1.0131x; 1.0006x over previous
import functools

import jax
import jax.numpy as jnp
from jax import lax
from jax.experimental import pallas as pl
from jax.experimental.pallas import tpu as pltpu
from jax.experimental.pallas import tpu_sc as plsc

NUM_CORES = 2
NUM_SUBCORES = 16
NUM_WORKERS = NUM_CORES * NUM_SUBCORES

S = 4096
T = 200
B = S * T
D = 64

STREAM = 100
GS = 4
GROUP = GS * T
G = GROUP // STREAM

S_PER_W = S // NUM_WORKERS
N_GROUPS = S_PER_W // GS
ROWS_PER_W = S_PER_W * T // STREAM

NBUF = 2

_mesh = plsc.VectorSubcoreMesh(core_axis_name="c", subcore_axis_name="s")


@functools.partial(
    pl.kernel,
    mesh=_mesh,
    out_type=jax.ShapeDtypeStruct((S, T, D), jnp.float32),
    scratch_types=[
        pltpu.VMEM((ROWS_PER_W, STREAM), jnp.int32),
        pltpu.VMEM((GS, T, D), jnp.float32),
        pltpu.VMEM((GS, T, D), jnp.float32),
        pltpu.SemaphoreType.DMA,
        pltpu.SemaphoreType.DMA,
        pltpu.SemaphoreType.DMA,
        pltpu.SemaphoreType.DMA,
    ],
    compiler_params=pltpu.CompilerParams(use_tc_tiling_on_sc=False),
)
def _emb_lookup(idx_hbm, table_hbm, out_hbm, idx_v, rows0, rows1,
                gat_sem0, gat_sem1, wb_sem0, wb_sem1):
    rows = (rows0, rows1)
    gat_sem = (gat_sem0, gat_sem1)
    wb_sem = (wb_sem0, wb_sem1)

    wid = lax.axis_index("s") * NUM_CORES + lax.axis_index("c")
    s_base = wid * S_PER_W

    pltpu.async_copy(
        idx_hbm.at[pl.ds(wid * ROWS_PER_W, ROWS_PER_W)], idx_v, gat_sem0
    ).wait()

    def streams(g, b):
        out = []
        for j in range(G):
            out.append((
                table_hbm.at[idx_v.at[g * G + j]],
                rows[b].at[j // 2, pl.ds((j % 2) * STREAM, STREAM)],
                gat_sem[b],
            ))
        return out

    def fire(g, b):
        for src, dst, sem in streams(g, b):
            pltpu.async_copy(src, dst, sem)

    def drain_gathers(g, b):
        for src, dst, sem in streams(g, b):
            pltpu.make_async_copy(src, dst, sem).wait()

    for b in range(NBUF):
        fire(b, b)

    def body(k, _):
        for b in range(NBUF):
            g_done = k * NBUF + b
            drain_gathers(g_done, b)
            pltpu.async_copy(
                rows[b], out_hbm.at[pl.ds(s_base + g_done * GS, GS)],
                wb_sem[b],
            ).wait()
            fire(g_done + NBUF, b)
        return ()

    lax.fori_loop(0, N_GROUPS // NBUF - 1, body, (), unroll=False)

    for b in range(NBUF):
        g_done = N_GROUPS - NBUF + b
        drain_gathers(g_done, b)
        pltpu.async_copy(
            rows[b], out_hbm.at[pl.ds(s_base + g_done * GS, GS)],
            wb_sem[b],
        ).wait()


def kernel(sentences_indices, table):
    idx2d = sentences_indices.reshape(B // STREAM, STREAM).astype(jnp.int32)
    return _emb_lookup(idx2d, table)
